# baseline (speedup 1.0000x reference)
import numpy as np
import jax
import jax.numpy as jnp
from jax.experimental import pallas as pl
from jax.experimental.pallas import tpu as pltpu

C = 96
H = 224
WD = 224
HW = H * WD
HP = H + 2
K = 32
ITERS = 3
NR = 8
P = NR * WD
NBLK = H // NR

_PREC = jax.lax.Precision.HIGHEST

_m = np.arange(C * 9)
_PERM = np.zeros((C * 9, C * 9), np.float32)
_PERM[_m, (_m % C) * 9 + (_m // C)] = 1.0


def _dot(a, b, ca, cb, prec=_PREC):
    return jax.lax.dot_general(
        a, b, (((ca,), (cb,)), ((), ())),
        preferred_element_type=jnp.float32, precision=prec)


def _dotb(a, b, ca, cb):
    return jax.lax.dot_general(
        a.astype(jnp.bfloat16), b.astype(jnp.bfloat16),
        (((ca,), (cb,)), ((), ())),
        preferred_element_type=jnp.float32)


def _blk(featp_ref, i, di, dj):
    v = featp_ref[pl.ds(i * NR + di, NR), pl.ds(dj, WD), :]
    return v.reshape(P, C)


def _min_mask(d):
    dmin = jnp.min(d, axis=1, keepdims=True)
    mask = d == dmin
    iota = jax.lax.broadcasted_iota(jnp.int32, d.shape, 1)
    amin = jnp.min(jnp.where(mask, iota, K), axis=1, keepdims=True)
    oh = (iota == amin).astype(jnp.float32)
    return oh, amin


def _dists(f, centers, cc):
    ff = jnp.sum(f * f, axis=1, keepdims=True)
    fc = _dot(f, centers, 1, 1, prec=jax.lax.Precision.DEFAULT)
    return ff - 2.0 * fc + cc[None, :]


def _kernel_a(x_ref, wg_ref, bg_ref, perm_ref,
              featp_ref, c2_ref, c3_ref, mod_ref):
    featp_ref[pl.ds(0, 1), :, :] = jnp.zeros((1, HP, C), jnp.float32)
    featp_ref[pl.ds(HP - 1, 1), :, :] = jnp.zeros((1, HP, C), jnp.float32)
    featp_ref[:, pl.ds(0, 1), :] = jnp.zeros((HP, 1, C), jnp.float32)
    featp_ref[:, pl.ds(HP - 1, 1), :] = jnp.zeros((HP, 1, C), jnp.float32)

    def pack(h, _):
        t = x_ref[:, pl.ds(h, 1), :].reshape(C, WD)
        featp_ref[pl.ds(h + 1, 1), pl.ds(1, WD), :] = t.T.reshape(1, WD, C)
        return 0

    jax.lax.fori_loop(0, H, pack, 0)

    ones = jnp.ones((P, 1), jnp.float32)
    centers = jnp.concatenate(
        [featp_ref[7 * k + 1, pl.ds(1, 1), :] for k in range(K)], axis=0)

    for t in range(ITERS):
        cc = jnp.sum(centers * centers, axis=1)
        last = t == ITERS - 1

        def body(i, carry, centers=centers, cc=cc, last=last):
            sums, counts, cps = carry
            f = _blk(featp_ref, i, 1, 1)
            oh, _ = _min_mask(_dists(f, centers, cc))
            sums = sums + _dot(oh, f, 0, 0)
            counts = counts + _dotb(oh, ones, 0, 0)
            if last:
                cps = tuple(
                    cps[j] + _dotb(oh, _blk(featp_ref, i, j // 3, j % 3), 0, 0)
                    for j in range(9))
            return sums, counts, cps

        z = jnp.zeros((K, C), jnp.float32)
        cps0 = tuple(z for _ in range(9)) if last else ()
        sums, counts, cps = jax.lax.fori_loop(
            0, NBLK, body, (z, jnp.zeros((K, 1), jnp.float32), cps0))
        counts = jnp.maximum(counts, 1.0)
        if last:
            c2_ref[...] = centers
            cpatch = jnp.concatenate(cps, axis=1) / counts
            perm = perm_ref[...]
            cpatch_i = _dotb(cpatch, perm, 1, 0)
            mod_i = jnp.tanh(_dotb(cpatch_i, wg_ref[...], 1, 0) + bg_ref[...])
            mod_ref[...] = _dotb(mod_i, perm, 1, 1)
        centers = sums / counts
    c3_ref[...] = centers


def _kernel_b(featp_ref, c2_ref, c3_ref, mod_ref, wr_ref, b_ref,
              y_ref, idx_ref, closs_ref):
    i = pl.program_id(0)
    c2 = c2_ref[...]
    cc2 = jnp.sum(c2 * c2, axis=1)
    f = _blk(featp_ref, i, 1, 1)
    d = _dists(f, c2, cc2)
    oh, _ = _min_mask(d)
    assign = jnp.argmin(d, axis=1).astype(jnp.int32)
    idx_ref[...] = assign.reshape(1, P // 128, 128)

    mod = mod_ref[...]
    acc = jnp.broadcast_to(b_ref[...], (P, C))
    for j in range(9):
        xs = _blk(featp_ref, i, j // 3, j % 3)
        mo = _dotb(oh, mod[:, j * C:(j + 1) * C], 1, 0)
        acc = acc + _dotb(xs * mo, wr_ref[j], 1, 0)

    res = jnp.where(acc >= 0, acc, 0.01 * acc)
    y_ref[...] = (f + res).T

    cg = _dotb(oh, c3_ref[...], 1, 0)
    diff = f - cg
    dsum = jnp.sum(diff * diff).reshape(1, 1)

    @pl.when(i == 0)
    def _():
        closs_ref[...] = jnp.zeros((1, 1), jnp.float32)

    closs_ref[...] += dsum

    @pl.when(i == NBLK - 1)
    def _():
        closs_ref[...] = closs_ref[...] / HW


def kernel(x, W, b, Wg, bg):
    x3 = x.reshape(C, H, WD)
    wr = W.reshape(C, C, 9).transpose(2, 1, 0)
    bg2 = bg.reshape(1, C * 9)
    b2 = b.reshape(1, C)
    perm = jnp.asarray(_PERM, jnp.bfloat16)

    featp, c2, c3, mod = pl.pallas_call(
        _kernel_a,
        out_shape=(
            jax.ShapeDtypeStruct((HP, HP, C), jnp.float32),
            jax.ShapeDtypeStruct((K, C), jnp.float32),
            jax.ShapeDtypeStruct((K, C), jnp.float32),
            jax.ShapeDtypeStruct((K, C * 9), jnp.float32),
        ),
    )(x3, Wg, bg2, perm)

    ycm, idx2d, closs = pl.pallas_call(
        _kernel_b,
        grid=(NBLK,),
        in_specs=[
            pl.BlockSpec((HP, HP, C), lambda i: (0, 0, 0)),
            pl.BlockSpec((K, C), lambda i: (0, 0)),
            pl.BlockSpec((K, C), lambda i: (0, 0)),
            pl.BlockSpec((K, C * 9), lambda i: (0, 0)),
            pl.BlockSpec((9, C, C), lambda i: (0, 0, 0)),
            pl.BlockSpec((1, C), lambda i: (0, 0)),
        ],
        out_specs=(
            pl.BlockSpec((C, P), lambda i: (0, i)),
            pl.BlockSpec((1, P // 128, 128), lambda i: (i, 0, 0)),
            pl.BlockSpec((1, 1), lambda i: (0, 0)),
        ),
        out_shape=(
            jax.ShapeDtypeStruct((C, HW), jnp.float32),
            jax.ShapeDtypeStruct((NBLK, P // 128, 128), jnp.int32),
            jax.ShapeDtypeStruct((1, 1), jnp.float32),
        ),
    )(featp, c2, c3, mod, wr, b2)

    y = ycm.reshape(1, C, H, WD)
    idx = idx2d.reshape(1, HW)
    return (y, idx, closs[0, 0])


# speedup vs baseline: 1.2368x; 1.2368x over previous
import numpy as np
import jax
import jax.numpy as jnp
from jax.experimental import pallas as pl
from jax.experimental.pallas import tpu as pltpu

C = 96
H = 224
WD = 224
HW = H * WD
HP = H + 2
K = 32
ITERS = 3
NR = 8
P = NR * WD
NBLK = H // NR

_PREC = jax.lax.Precision.HIGHEST

_m = np.arange(C * 9)
_PERM = np.zeros((C * 9, C * 9), np.float32)
_PERM[_m, (_m % C) * 9 + (_m // C)] = 1.0


def _dot(a, b, ca, cb, prec=_PREC):
    return jax.lax.dot_general(
        a, b, (((ca,), (cb,)), ((), ())),
        preferred_element_type=jnp.float32, precision=prec)


def _dotb(a, b, ca, cb):
    return jax.lax.dot_general(
        a.astype(jnp.bfloat16), b.astype(jnp.bfloat16),
        (((ca,), (cb,)), ((), ())),
        preferred_element_type=jnp.float32)


def _blk(featp_ref, i, di, dj):
    v = featp_ref[pl.ds(i * NR + di, NR), pl.ds(dj, WD), :]
    return v.reshape(P, C)


def _min_mask(d):
    dmin = jnp.min(d, axis=1, keepdims=True)
    mask = d == dmin
    iota = jax.lax.broadcasted_iota(jnp.int32, d.shape, 1)
    amin = jnp.min(jnp.where(mask, iota, K), axis=1, keepdims=True)
    oh = (iota == amin).astype(jnp.float32)
    return oh, amin


def _dists(f, centers, cc):
    ff = jnp.sum(f * f, axis=1, keepdims=True)
    fc = _dot(f, centers, 1, 1, prec=jax.lax.Precision.DEFAULT)
    return ff - 2.0 * fc + cc[None, :]


def _kernel_a(x_ref, wg_ref, bg_ref, perm_ref,
              featp_ref, c2_ref, c3_ref, mod_ref):
    featp_ref[pl.ds(0, 1), :, :] = jnp.zeros((1, HP, C), jnp.float32)
    featp_ref[pl.ds(HP - 1, 1), :, :] = jnp.zeros((1, HP, C), jnp.float32)
    featp_ref[:, pl.ds(0, 1), :] = jnp.zeros((HP, 1, C), jnp.float32)
    featp_ref[:, pl.ds(HP - 1, 1), :] = jnp.zeros((HP, 1, C), jnp.float32)

    def pack(h, _):
        t = x_ref[:, pl.ds(h, 1), :].reshape(C, WD)
        featp_ref[pl.ds(h + 1, 1), pl.ds(1, WD), :] = t.T.reshape(1, WD, C)
        return 0

    jax.lax.fori_loop(0, H, pack, 0)

    ones = jnp.ones((P, 1), jnp.float32)
    centers = jnp.concatenate(
        [featp_ref[7 * k + 1, pl.ds(1, 1), :] for k in range(K)], axis=0)

    for t in range(ITERS):
        cc = jnp.sum(centers * centers, axis=1)
        last = t == ITERS - 1

        def body(i, carry, centers=centers, cc=cc, last=last):
            sums, counts, cps = carry
            f = _blk(featp_ref, i, 1, 1)
            oh, _ = _min_mask(_dists(f, centers, cc))
            sums = sums + _dot(oh, f, 0, 0)
            counts = counts + _dot(oh, ones, 0, 0)
            if last:
                cps = tuple(
                    cps[j] + _dotb(oh, _blk(featp_ref, i, j // 3, j % 3), 0, 0)
                    for j in range(9))
            return sums, counts, cps

        z = jnp.zeros((K, C), jnp.float32)
        cps0 = tuple(z for _ in range(9)) if last else ()
        sums, counts, cps = jax.lax.fori_loop(
            0, NBLK, body, (z, jnp.zeros((K, 1), jnp.float32), cps0))
        counts = jnp.maximum(counts, 1.0)
        if last:
            c2_ref[...] = centers
            cpatch = jnp.concatenate(cps, axis=1) / counts
            perm = perm_ref[...]
            cpatch_i = _dotb(cpatch, perm, 1, 0)
            mod_i = jnp.tanh(_dotb(cpatch_i, wg_ref[...], 1, 0) + bg_ref[...])
            mod_ref[...] = _dotb(mod_i, perm, 1, 1)
        centers = sums / counts
    c3_ref[...] = centers


def _kernel_b(featp_ref, c2_ref, c3_ref, mod_ref, wr_ref, b_ref,
              y_ref, idx_ref, closs_ref):
    i = pl.program_id(0)
    c2 = c2_ref[...]
    cc2 = jnp.sum(c2 * c2, axis=1)
    f = _blk(featp_ref, i, 1, 1)
    d = _dists(f, c2, cc2)
    oh, _ = _min_mask(d)
    assign = jnp.argmin(d, axis=1).astype(jnp.int32)
    idx_ref[...] = assign.reshape(1, P // 128, 128)

    mod = mod_ref[...]
    acc = jnp.broadcast_to(b_ref[...], (P, C))
    for j in range(9):
        xs = _blk(featp_ref, i, j // 3, j % 3)
        mo = _dotb(oh, mod[:, j * C:(j + 1) * C], 1, 0)
        acc = acc + _dotb(xs * mo, wr_ref[j], 1, 0)

    res = jnp.where(acc >= 0, acc, 0.01 * acc)
    y_ref[...] = (f + res).T

    cg = _dotb(oh, c3_ref[...], 1, 0)
    diff = f - cg
    dsum = jnp.sum(diff * diff).reshape(1, 1)

    @pl.when(i == 0)
    def _():
        closs_ref[...] = jnp.zeros((1, 1), jnp.float32)

    closs_ref[...] += dsum

    @pl.when(i == NBLK - 1)
    def _():
        closs_ref[...] = closs_ref[...] / HW


def kernel(x, W, b, Wg, bg):
    x3 = x.reshape(C, H, WD)
    wr = W.reshape(C, C, 9).transpose(2, 1, 0)
    bg2 = bg.reshape(1, C * 9)
    b2 = b.reshape(1, C)
    perm = jnp.asarray(_PERM, jnp.bfloat16)

    featp, c2, c3, mod = pl.pallas_call(
        _kernel_a,
        out_shape=(
            jax.ShapeDtypeStruct((HP, HP, C), jnp.float32),
            jax.ShapeDtypeStruct((K, C), jnp.float32),
            jax.ShapeDtypeStruct((K, C), jnp.float32),
            jax.ShapeDtypeStruct((K, C * 9), jnp.float32),
        ),
    )(x3, Wg, bg2, perm)

    ycm, idx2d, closs = pl.pallas_call(
        _kernel_b,
        grid=(NBLK,),
        in_specs=[
            pl.BlockSpec((HP, HP, C), lambda i: (0, 0, 0)),
            pl.BlockSpec((K, C), lambda i: (0, 0)),
            pl.BlockSpec((K, C), lambda i: (0, 0)),
            pl.BlockSpec((K, C * 9), lambda i: (0, 0)),
            pl.BlockSpec((9, C, C), lambda i: (0, 0, 0)),
            pl.BlockSpec((1, C), lambda i: (0, 0)),
        ],
        out_specs=(
            pl.BlockSpec((C, P), lambda i: (0, i)),
            pl.BlockSpec((1, P // 128, 128), lambda i: (i, 0, 0)),
            pl.BlockSpec((1, 1), lambda i: (0, 0)),
        ),
        out_shape=(
            jax.ShapeDtypeStruct((C, HW), jnp.float32),
            jax.ShapeDtypeStruct((NBLK, P // 128, 128), jnp.int32),
            jax.ShapeDtypeStruct((1, 1), jnp.float32),
        ),
    )(featp, c2, c3, mod, wr, b2)

    y = ycm.reshape(1, C, H, WD)
    idx = idx2d.reshape(1, HW)
    return (y, idx, closs[0, 0])


# speedup vs baseline: 1.4111x; 1.1409x over previous
import numpy as np
import jax
import jax.numpy as jnp
from jax.experimental import pallas as pl
from jax.experimental.pallas import tpu as pltpu

C = 96
H = 224
WD = 224
HW = H * WD
HP = H + 2
K = 32
ITERS = 3
NR = 8
P = NR * WD
NBLK = H // NR

_PREC = jax.lax.Precision.HIGHEST

_m = np.arange(C * 9)
_PERM = np.zeros((C * 9, C * 9), np.float32)
_PERM[_m, (_m % C) * 9 + (_m // C)] = 1.0


def _dot(a, b, ca, cb, prec=_PREC):
    return jax.lax.dot_general(
        a, b, (((ca,), (cb,)), ((), ())),
        preferred_element_type=jnp.float32, precision=prec)


def _dotb(a, b, ca, cb):
    return jax.lax.dot_general(
        a.astype(jnp.bfloat16), b.astype(jnp.bfloat16),
        (((ca,), (cb,)), ((), ())),
        preferred_element_type=jnp.float32)


def _segdot(oh, f):
    hi = f.astype(jnp.bfloat16)
    r1 = f - hi.astype(jnp.float32)
    mid = r1.astype(jnp.bfloat16)
    lo = (r1 - mid.astype(jnp.float32)).astype(jnp.bfloat16)
    ohb = oh.astype(jnp.bfloat16)

    def g(t):
        return jax.lax.dot_general(
            ohb, t, (((0,), (0,)), ((), ())),
            preferred_element_type=jnp.float32)

    return g(hi) + g(mid) + g(lo)


def _blk(featp_ref, i, di, dj):
    v = featp_ref[pl.ds(i * NR + di, NR), pl.ds(dj, WD), :]
    return v.reshape(P, C)


def _min_mask(d):
    dmin = jnp.min(d, axis=1, keepdims=True)
    mask = d == dmin
    iota = jax.lax.broadcasted_iota(jnp.int32, d.shape, 1)
    amin = jnp.min(jnp.where(mask, iota, K), axis=1, keepdims=True)
    oh = (iota == amin).astype(jnp.float32)
    return oh, amin


def _dists(f, centers, cc):
    ff = jnp.sum(f * f, axis=1, keepdims=True)
    fc = _dot(f, centers, 1, 1, prec=jax.lax.Precision.DEFAULT)
    return ff - 2.0 * fc + cc[None, :]


def _kernel_a(x_ref, wg_ref, bg_ref, perm_ref,
              featp_ref, c2_ref, c3_ref, mod_ref):
    featp_ref[pl.ds(0, 1), :, :] = jnp.zeros((1, HP, C), jnp.float32)
    featp_ref[pl.ds(HP - 1, 1), :, :] = jnp.zeros((1, HP, C), jnp.float32)
    featp_ref[:, pl.ds(0, 1), :] = jnp.zeros((HP, 1, C), jnp.float32)
    featp_ref[:, pl.ds(HP - 1, 1), :] = jnp.zeros((HP, 1, C), jnp.float32)

    def pack(h, _):
        t = x_ref[:, pl.ds(h, 1), :].reshape(C, WD)
        featp_ref[pl.ds(h + 1, 1), pl.ds(1, WD), :] = t.T.reshape(1, WD, C)
        return 0

    jax.lax.fori_loop(0, H, pack, 0)

    ones = jnp.ones((P, 1), jnp.float32)
    centers = jnp.concatenate(
        [featp_ref[7 * k + 1, pl.ds(1, 1), :] for k in range(K)], axis=0)

    for t in range(ITERS):
        cc = jnp.sum(centers * centers, axis=1)
        last = t == ITERS - 1

        def body(i, carry, centers=centers, cc=cc, last=last):
            sums, cps = carry
            f = _blk(featp_ref, i, 1, 1)
            oh, _ = _min_mask(_dists(f, centers, cc))
            sums = sums + _segdot(oh, jnp.concatenate([f, ones], axis=1))
            if last:
                cps = tuple(
                    cps[j] + _dotb(oh, _blk(featp_ref, i, j // 3, j % 3), 0, 0)
                    for j in range(9))
            return sums, cps

        z = jnp.zeros((K, C), jnp.float32)
        cps0 = tuple(z for _ in range(9)) if last else ()
        sums_ext, cps = jax.lax.fori_loop(
            0, NBLK, body, (jnp.zeros((K, C + 1), jnp.float32), cps0))
        sums = sums_ext[:, :C]
        counts = jnp.maximum(sums_ext[:, C:], 1.0)
        if last:
            c2_ref[...] = centers
            cpatch = jnp.concatenate(cps, axis=1) / counts
            perm = perm_ref[...]
            cpatch_i = _dotb(cpatch, perm, 1, 0)
            mod_i = jnp.tanh(_dotb(cpatch_i, wg_ref[...], 1, 0) + bg_ref[...])
            mod_ref[...] = _dotb(mod_i, perm, 1, 1)
        centers = sums / counts
    c3_ref[...] = centers


def _kernel_b(featp_ref, c2_ref, c3_ref, mod_ref, wr_ref, b_ref,
              y_ref, idx_ref, closs_ref):
    i = pl.program_id(0)
    c2 = c2_ref[...]
    cc2 = jnp.sum(c2 * c2, axis=1)
    f = _blk(featp_ref, i, 1, 1)
    d = _dists(f, c2, cc2)
    oh, _ = _min_mask(d)
    assign = jnp.argmin(d, axis=1).astype(jnp.int32)
    idx_ref[...] = assign.reshape(1, P // 128, 128)

    mod = mod_ref[...]
    acc = jnp.broadcast_to(b_ref[...], (P, C))
    for j in range(9):
        xs = _blk(featp_ref, i, j // 3, j % 3)
        mo = _dotb(oh, mod[:, j * C:(j + 1) * C], 1, 0)
        acc = acc + _dotb(xs * mo, wr_ref[j], 1, 0)

    res = jnp.where(acc >= 0, acc, 0.01 * acc)
    y_ref[...] = (f + res).T

    cg = _dotb(oh, c3_ref[...], 1, 0)
    diff = f - cg
    dsum = jnp.sum(diff * diff).reshape(1, 1)

    @pl.when(i == 0)
    def _():
        closs_ref[...] = jnp.zeros((1, 1), jnp.float32)

    closs_ref[...] += dsum

    @pl.when(i == NBLK - 1)
    def _():
        closs_ref[...] = closs_ref[...] / HW


def kernel(x, W, b, Wg, bg):
    x3 = x.reshape(C, H, WD)
    wr = W.reshape(C, C, 9).transpose(2, 1, 0)
    bg2 = bg.reshape(1, C * 9)
    b2 = b.reshape(1, C)
    perm = jnp.asarray(_PERM, jnp.bfloat16)

    featp, c2, c3, mod = pl.pallas_call(
        _kernel_a,
        out_shape=(
            jax.ShapeDtypeStruct((HP, HP, C), jnp.float32),
            jax.ShapeDtypeStruct((K, C), jnp.float32),
            jax.ShapeDtypeStruct((K, C), jnp.float32),
            jax.ShapeDtypeStruct((K, C * 9), jnp.float32),
        ),
    )(x3, Wg, bg2, perm)

    ycm, idx2d, closs = pl.pallas_call(
        _kernel_b,
        grid=(NBLK,),
        in_specs=[
            pl.BlockSpec((HP, HP, C), lambda i: (0, 0, 0)),
            pl.BlockSpec((K, C), lambda i: (0, 0)),
            pl.BlockSpec((K, C), lambda i: (0, 0)),
            pl.BlockSpec((K, C * 9), lambda i: (0, 0)),
            pl.BlockSpec((9, C, C), lambda i: (0, 0, 0)),
            pl.BlockSpec((1, C), lambda i: (0, 0)),
        ],
        out_specs=(
            pl.BlockSpec((C, P), lambda i: (0, i)),
            pl.BlockSpec((1, P // 128, 128), lambda i: (i, 0, 0)),
            pl.BlockSpec((1, 1), lambda i: (0, 0)),
        ),
        out_shape=(
            jax.ShapeDtypeStruct((C, HW), jnp.float32),
            jax.ShapeDtypeStruct((NBLK, P // 128, 128), jnp.int32),
            jax.ShapeDtypeStruct((1, 1), jnp.float32),
        ),
    )(featp, c2, c3, mod, wr, b2)

    y = ycm.reshape(1, C, H, WD)
    idx = idx2d.reshape(1, HW)
    return (y, idx, closs[0, 0])


# speedup vs baseline: 1.8478x; 1.3094x over previous
import numpy as np
import jax
import jax.numpy as jnp
from jax.experimental import pallas as pl
from jax.experimental.pallas import tpu as pltpu

C = 96
H = 224
WD = 224
HW = H * WD
HP = H + 2
K = 32
ITERS = 3
NR = 8
P = NR * WD
NBLK = H // NR

_m = np.arange(C * 9)
_PERM = np.zeros((C * 9, C * 9), np.float32)
_PERM[_m, (_m % C) * 9 + (_m // C)] = 1.0


def _dot(a, b, ca, cb, prec=jax.lax.Precision.DEFAULT):
    return jax.lax.dot_general(
        a, b, (((ca,), (cb,)), ((), ())),
        preferred_element_type=jnp.float32, precision=prec)


def _dotb(a, b, ca, cb):
    return jax.lax.dot_general(
        a.astype(jnp.bfloat16), b.astype(jnp.bfloat16),
        (((ca,), (cb,)), ((), ())),
        preferred_element_type=jnp.float32)


def _segdot_t(fext_t, oht):
    hi = fext_t.astype(jnp.bfloat16)
    r1 = fext_t - hi.astype(jnp.float32)
    mid = r1.astype(jnp.bfloat16)
    lo = (r1 - mid.astype(jnp.float32)).astype(jnp.bfloat16)
    ohb = oht.astype(jnp.bfloat16)

    def g(t):
        return jax.lax.dot_general(
            t, ohb, (((1,), (1,)), ((), ())),
            preferred_element_type=jnp.float32)

    return g(hi) + g(mid) + g(lo)


def _blk(featp_ref, i, di, dj):
    v = featp_ref[pl.ds(i * NR + di, NR), pl.ds(dj, WD), :]
    return v.reshape(P, C)


def _min_mask_t(dt):
    dmin = jnp.min(dt, axis=0, keepdims=True)
    mask = dt == dmin
    iota = jax.lax.broadcasted_iota(jnp.int32, dt.shape, 0)
    amin = jnp.min(jnp.where(mask, iota, K), axis=0, keepdims=True)
    return (iota == amin).astype(jnp.float32)


def _dists_t(ft, ct, cc_col):
    fft = jnp.sum(ft * ft, axis=0, keepdims=True)
    fct = _dot(ct, ft, 0, 0)
    return (fft - 2.0 * fct) + cc_col


def _kernel_a(xf_ref, wg_ref, bg_ref, perm_ref,
              featp_ref, c2t_ref, c3_ref, mod_ref):
    featp_ref[pl.ds(0, 1), :, :] = jnp.zeros((1, HP, C), jnp.float32)
    featp_ref[pl.ds(HP - 1, 1), :, :] = jnp.zeros((1, HP, C), jnp.float32)
    featp_ref[:, pl.ds(0, 1), :] = jnp.zeros((HP, 1, C), jnp.float32)
    featp_ref[:, pl.ds(HP - 1, 1), :] = jnp.zeros((HP, 1, C), jnp.float32)

    def pack(g, _):
        tt = xf_ref[:, pl.ds(g * 4 * WD, 4 * WD)].T
        for r in range(4):
            featp_ref[pl.ds(g * 4 + r + 1, 1), pl.ds(1, WD), :] = \
                tt[r * WD:(r + 1) * WD, :].reshape(1, WD, C)
        return 0

    jax.lax.fori_loop(0, H // 4, pack, 0)

    ones_t = jnp.ones((1, P), jnp.float32)
    ct = jnp.concatenate(
        [xf_ref[:, pl.ds(k * (HW // K), 1)] for k in range(K)], axis=1)

    for t in range(ITERS):
        cc_col = jnp.sum(ct * ct, axis=0)[:, None]
        last = t == ITERS - 1

        def body(i, carry, ct=ct, cc_col=cc_col, last=last):
            sums, cps = carry
            ft = xf_ref[:, pl.ds(i * P, P)]
            oht = _min_mask_t(_dists_t(ft, ct, cc_col))
            fext_t = jnp.concatenate([ft, ones_t], axis=0)
            sums = sums + _segdot_t(fext_t, oht)
            if last:
                cps = tuple(
                    cps[j] + _dotb(oht, _blk(featp_ref, i, j // 3, j % 3),
                                   1, 0)
                    for j in range(9))
            return sums, cps

        z = jnp.zeros((K, C), jnp.float32)
        cps0 = tuple(z for _ in range(9)) if last else ()
        sums_ext, cps = jax.lax.fori_loop(
            0, NBLK, body, (jnp.zeros((C + 1, K), jnp.float32), cps0))
        counts_row = jnp.maximum(sums_ext[C:, :], 1.0)
        if last:
            c2t_ref[...] = ct
            cpatch = jnp.concatenate(cps, axis=1) / counts_row.T
            perm = perm_ref[...]
            cpatch_i = _dotb(cpatch, perm, 1, 0)
            mod_i = jnp.tanh(_dotb(cpatch_i, wg_ref[...], 1, 0) + bg_ref[...])
            mod_ref[...] = _dotb(mod_i, perm, 1, 1)
        ct = sums_ext[:C, :] / counts_row
    c3_ref[...] = ct.T


def _kernel_b(featp_ref, xf_ref, c2t_ref, c3_ref, mod_ref, wr_ref, b_ref,
              y_ref, idx_ref, closs_ref):
    i = pl.program_id(0)
    c2t = c2t_ref[...]
    cc_col = jnp.sum(c2t * c2t, axis=0)[:, None]
    ft = xf_ref[:, pl.ds(i * P, P)]
    dt = _dists_t(ft, c2t, cc_col)
    oht = _min_mask_t(dt)
    assign = jnp.argmin(dt, axis=0).astype(jnp.int32)
    idx_ref[...] = assign.reshape(1, P // 128, 128)

    f = _blk(featp_ref, i, 1, 1)
    mod = mod_ref[...]
    acc = jnp.broadcast_to(b_ref[...], (P, C))
    for j in range(9):
        xs = _blk(featp_ref, i, j // 3, j % 3)
        mo = _dotb(oht, mod[:, j * C:(j + 1) * C], 0, 0)
        acc = acc + _dotb(xs * mo, wr_ref[j], 1, 0)

    res = jnp.where(acc >= 0, acc, 0.01 * acc)
    y_ref[...] = (f + res).T

    cg = _dotb(oht, c3_ref[...], 0, 0)
    diff = f - cg
    dsum = jnp.sum(diff * diff).reshape(1, 1)

    @pl.when(i == 0)
    def _():
        closs_ref[...] = jnp.zeros((1, 1), jnp.float32)

    closs_ref[...] += dsum

    @pl.when(i == NBLK - 1)
    def _():
        closs_ref[...] = closs_ref[...] / HW


def kernel(x, W, b, Wg, bg):
    xf = x.reshape(C, HW)
    wr = W.reshape(C, C, 9).transpose(2, 1, 0)
    bg2 = bg.reshape(1, C * 9)
    b2 = b.reshape(1, C)
    perm = jnp.asarray(_PERM, jnp.bfloat16)

    featp, c2t, c3, mod = pl.pallas_call(
        _kernel_a,
        out_shape=(
            jax.ShapeDtypeStruct((HP, HP, C), jnp.float32),
            jax.ShapeDtypeStruct((C, K), jnp.float32),
            jax.ShapeDtypeStruct((K, C), jnp.float32),
            jax.ShapeDtypeStruct((K, C * 9), jnp.float32),
        ),
    )(xf, Wg, bg2, perm)

    ycm, idx2d, closs = pl.pallas_call(
        _kernel_b,
        grid=(NBLK,),
        in_specs=[
            pl.BlockSpec((HP, HP, C), lambda i: (0, 0, 0)),
            pl.BlockSpec((C, HW), lambda i: (0, 0)),
            pl.BlockSpec((C, K), lambda i: (0, 0)),
            pl.BlockSpec((K, C), lambda i: (0, 0)),
            pl.BlockSpec((K, C * 9), lambda i: (0, 0)),
            pl.BlockSpec((9, C, C), lambda i: (0, 0, 0)),
            pl.BlockSpec((1, C), lambda i: (0, 0)),
        ],
        out_specs=(
            pl.BlockSpec((C, P), lambda i: (0, i)),
            pl.BlockSpec((1, P // 128, 128), lambda i: (i, 0, 0)),
            pl.BlockSpec((1, 1), lambda i: (0, 0)),
        ),
        out_shape=(
            jax.ShapeDtypeStruct((C, HW), jnp.float32),
            jax.ShapeDtypeStruct((NBLK, P // 128, 128), jnp.int32),
            jax.ShapeDtypeStruct((1, 1), jnp.float32),
        ),
    )(featp, xf, c2t, c3, mod, wr, b2)

    y = ycm.reshape(1, C, H, WD)
    idx = idx2d.reshape(1, HW)
    return (y, idx, closs[0, 0])
